# baseline (speedup 1.0000x reference)
import functools

import jax
import jax.numpy as jnp
from jax import lax
from jax.experimental import pallas as pl
from jax.experimental.pallas import tpu as pltpu
from jax.experimental.pallas import tpu_sc as plsc

_N = 10000
_E = 320000
_G = 128
_DOUT = 10

_NTILES = 16
_NPAD = 10240
_NPT = _NPAD // _NTILES
_ECR = _E // _NTILES
_RS = 10
_CH = 2048
_ECP = _RS * _CH
_L = 16


def _rsqrt16(x):
    i = plsc.bitcast(x, jnp.int32)
    i = 0x5F3759DF - lax.shift_right_arithmetic(i, 1)
    y = plsc.bitcast(i, jnp.float32)
    for _ in range(3):
        y = y * (1.5 - 0.5 * x * y * y)
    return y


def _sc_compute(x_pad, srcp, dstp):
    mesh = plsc.VectorSubcoreMesh(
        core_axis_name="c", subcore_axis_name="s", num_cores=1,
        num_subcores=_NTILES)

    @functools.partial(
        pl.kernel,
        out_type=(
            jax.ShapeDtypeStruct((_NPAD,), jnp.float32),
            jax.ShapeDtypeStruct((_NPAD,), jnp.float32),
        ),
        mesh=mesh,
        scratch_types=[
            pltpu.VMEM((_RS, 1, _CH), jnp.int32),
            pltpu.VMEM((_RS, 1, _CH), jnp.int32),
            pltpu.VMEM((_RS, 1, _CH), jnp.float32),
            pltpu.VMEM((_RS, 1, _CH), jnp.float32),
            pltpu.VMEM((_NPAD,), jnp.float32),
            pltpu.VMEM((_NPAD,), jnp.float32),
            pltpu.VMEM((_NPAD,), jnp.float32),
            pltpu.VMEM((_NPT,), jnp.float32),
            pltpu.VMEM_SHARED((_NPAD,), jnp.float32),
            pltpu.VMEM_SHARED((_NPAD,), jnp.float32),
        ],
        compiler_params=pltpu.CompilerParams(needs_layout_passes=False),
    )
    def sc_kernel(x_hbm, src_hbm, dst_hbm, t1_hbm, t2_hbm,
                  src_v, dst_v, w_v, val_v, x_v, dinv_v, t1_v, sl_v,
                  acc_sh, aux_sh):
        tid = lax.axis_index("s")
        nsl = pl.ds(tid * _NPT, _NPT)
        ebase = tid * _ECP

        for r in range(_RS):
            pltpu.sync_copy(src_hbm.at[pl.ds(ebase + r * _CH, _CH)],
                            src_v.at[r, 0])
            pltpu.sync_copy(dst_hbm.at[pl.ds(ebase + r * _CH, _CH)],
                            dst_v.at[r, 0])
        pltpu.sync_copy(x_hbm, x_v)

        def _zero_sl(i, _):
            sl_v[pl.ds(i * _L, _L)] = jnp.zeros((_L,), jnp.float32)
            return 0
        lax.fori_loop(0, _NPT // _L, _zero_sl, 0)
        pltpu.sync_copy(sl_v, acc_sh.at[nsl])

        def _ones(j, _):
            r = j // (_CH // _L)
            c = j % (_CH // _L)
            w_v[r, 0, pl.ds(c * _L, _L)] = jnp.ones((_L,), jnp.float32)
            return 0
        lax.fori_loop(0, _RS * (_CH // _L), _ones, 0)

        plsc.subcore_barrier()

        for r in range(_RS):
            pltpu.sync_copy(w_v.at[r, 0], acc_sh.at[src_v.at[r, 0]], add=True)

        plsc.subcore_barrier()

        pltpu.sync_copy(acc_sh.at[nsl], sl_v)

        def _dinv(i, _):
            d = sl_v[pl.ds(i * _L, _L)]
            y = _rsqrt16(d)
            sl_v[pl.ds(i * _L, _L)] = jnp.where(d > 0.5, y, 0.0)
            return 0
        lax.fori_loop(0, _NPT // _L, _dinv, 0)
        pltpu.sync_copy(sl_v, aux_sh.at[nsl])
        lax.fori_loop(0, _NPT // _L, _zero_sl, 0)
        pltpu.sync_copy(sl_v, acc_sh.at[nsl])

        plsc.subcore_barrier()

        pltpu.sync_copy(aux_sh, dinv_v)

        def _wval(j, _):
            r = j // (_CH // _L)
            c = j % (_CH // _L)
            cs = pl.ds(c * _L, _L)
            s16 = src_v[r, 0, cs]
            d16 = dst_v[r, 0, cs]
            dvs = plsc.load_gather(dinv_v, [s16])
            dvd = plsc.load_gather(dinv_v, [d16])
            x16 = plsc.load_gather(x_v, [s16])
            w16 = -(dvs * dvd)
            w_v[r, 0, cs] = w16
            val_v[r, 0, cs] = w16 * x16
            return 0
        lax.fori_loop(0, _RS * (_CH // _L), _wval, 0)

        for r in range(_RS):
            pltpu.sync_copy(val_v.at[r, 0], acc_sh.at[dst_v.at[r, 0]], add=True)

        plsc.subcore_barrier()

        pltpu.sync_copy(acc_sh, t1_v)
        pltpu.sync_copy(acc_sh.at[nsl], t1_hbm.at[nsl])
        lax.fori_loop(0, _NPT // _L, _zero_sl, 0)
        pltpu.sync_copy(sl_v, aux_sh.at[nsl])

        plsc.subcore_barrier()

        def _val2(j, _):
            r = j // (_CH // _L)
            c = j % (_CH // _L)
            cs = pl.ds(c * _L, _L)
            s16 = src_v[r, 0, cs]
            t16 = plsc.load_gather(t1_v, [s16])
            val_v[r, 0, cs] = 2.0 * w_v[r, 0, cs] * t16
            return 0
        lax.fori_loop(0, _RS * (_CH // _L), _val2, 0)
        for r in range(_RS):
            pltpu.sync_copy(val_v.at[r, 0], aux_sh.at[dst_v.at[r, 0]], add=True)

        plsc.subcore_barrier()

        pltpu.sync_copy(aux_sh.at[nsl], sl_v)

        def _t2(i, _):
            ds = pl.ds(tid * _NPT + i * _L, _L)
            sl_v[pl.ds(i * _L, _L)] = sl_v[pl.ds(i * _L, _L)] - x_v[ds]
            return 0
        lax.fori_loop(0, _NPT // _L, _t2, 0)
        pltpu.sync_copy(sl_v, t2_hbm.at[nsl])

    return sc_kernel(x_pad, srcp, dstp)


_NB = 400
_NBLK = _N // _NB


def _tc_body(t_ref, wr_ref, c_ref, bcb_ref, bfc_ref, out_ref, acc_ref):
    j = pl.program_id(0)

    @pl.when(j == 0)
    def _():
        acc_ref[...] = jnp.zeros_like(acc_ref)

    tb = t_ref[...]
    h = jnp.dot(tb, c_ref[...], preferred_element_type=jnp.float32)
    h = jnp.maximum(h + bcb_ref[...], 0.0)
    w3 = wr_ref[...]
    for d in range(_DOUT):
        acc_ref[d:d + 1, :] += jnp.sum(w3[d] * h, axis=0, keepdims=True)

    @pl.when(j == _NBLK - 1)
    def _():
        s = jnp.sum(acc_ref[0:_DOUT, :], axis=1, keepdims=True)
        y = s + bfc_ref[...]
        m = jnp.max(y, axis=0, keepdims=True)
        z = y - m
        lse = jnp.log(jnp.sum(jnp.exp(z), axis=0, keepdims=True))
        out_ref[...] = z - lse


def kernel(x, edge_index, W_cheb, b_cheb, W_fc, b_fc):
    x1 = x[:, 0]
    x_pad = jnp.pad(x1, (0, _NPAD - _N))

    padvec = _N + (jnp.arange(_NTILES * _ECP - _E, dtype=jnp.int32)
                   % (_NPAD - _N))
    srcg = jnp.concatenate([edge_index[0], padvec])
    dstg = jnp.concatenate([edge_index[1], padvec])

    t1, t2 = _sc_compute(x_pad, srcg, dstg)

    tmat = jnp.concatenate(
        [x, t1[:_N, None], t2[:_N, None],
         jnp.zeros((_N, 5), jnp.float32)], axis=1)
    wr = W_fc.reshape(_DOUT, _N, _G)
    cmat = jnp.zeros((8, _G), jnp.float32).at[:3].set(W_cheb.reshape(3, _G))
    bcb = b_cheb.reshape(1, _G)
    bfc = b_fc.reshape(_DOUT, 1)

    out = pl.pallas_call(
        _tc_body,
        grid=(_NBLK,),
        in_specs=[
            pl.BlockSpec((_NB, 8), lambda j: (j, 0)),
            pl.BlockSpec((_DOUT, _NB, _G), lambda j: (0, j, 0)),
            pl.BlockSpec((8, _G), lambda j: (0, 0)),
            pl.BlockSpec((1, _G), lambda j: (0, 0)),
            pl.BlockSpec((_DOUT, 1), lambda j: (0, 0)),
        ],
        out_specs=pl.BlockSpec((_DOUT, 1), lambda j: (0, 0)),
        out_shape=jax.ShapeDtypeStruct((_DOUT, 1), jnp.float32),
        scratch_shapes=[pltpu.VMEM((16, _G), jnp.float32)],
        compiler_params=pltpu.CompilerParams(
            dimension_semantics=("arbitrary",)),
    )(tmat, wr, cmat, bcb, bfc)
    return out[:, 0]


# speedup vs baseline: 1.0072x; 1.0072x over previous
import functools

import jax
import jax.numpy as jnp
from jax import lax
from jax.experimental import pallas as pl
from jax.experimental.pallas import tpu as pltpu
from jax.experimental.pallas import tpu_sc as plsc

_N = 10000
_E = 320000
_G = 128
_DOUT = 10

_NTILES = 16
_NPAD = 10240
_NPT = _NPAD // _NTILES
_ECR = _E // _NTILES
_RS = 10
_CH = 2048
_ECP = _RS * _CH
_L = 16


def _rsqrt16(x):
    i = plsc.bitcast(x, jnp.int32)
    i = 0x5F3759DF - lax.shift_right_arithmetic(i, 1)
    y = plsc.bitcast(i, jnp.float32)
    for _ in range(3):
        y = y * (1.5 - 0.5 * x * y * y)
    return y


def _sc_compute(x_pad, ei):
    mesh = plsc.VectorSubcoreMesh(
        core_axis_name="c", subcore_axis_name="s", num_cores=1,
        num_subcores=_NTILES)

    @functools.partial(
        pl.kernel,
        out_type=(
            jax.ShapeDtypeStruct((_NPAD,), jnp.float32),
            jax.ShapeDtypeStruct((_NPAD,), jnp.float32),
        ),
        mesh=mesh,
        scratch_types=[
            pltpu.VMEM((_RS, 1, _CH), jnp.int32),
            pltpu.VMEM((_RS, 1, _CH), jnp.int32),
            pltpu.VMEM((_RS, 1, _CH), jnp.float32),
            pltpu.VMEM((_RS, 1, _CH), jnp.float32),
            pltpu.VMEM((_NPAD,), jnp.float32),
            pltpu.VMEM((_NPAD,), jnp.float32),
            pltpu.VMEM((_NPAD,), jnp.float32),
            pltpu.VMEM((_NPT,), jnp.float32),
            pltpu.VMEM_SHARED((_NPAD,), jnp.float32),
            pltpu.VMEM_SHARED((_NPAD,), jnp.float32),
        ],
        compiler_params=pltpu.CompilerParams(needs_layout_passes=False),
    )
    def sc_kernel(x_hbm, ei_hbm, t1_hbm, t2_hbm,
                  src_v, dst_v, w_v, val_v, x_v, dinv_v, t1_v, sl_v,
                  acc_sh, aux_sh):
        tid = lax.axis_index("s")
        nsl = pl.ds(tid * _NPT, _NPT)
        ebase = tid * _ECP

        last = tid == _NTILES - 1

        @pl.when(jnp.logical_not(last))
        def _():
            for r in range(_RS):
                pltpu.sync_copy(ei_hbm.at[0, pl.ds(ebase + r * _CH, _CH)],
                                src_v.at[r, 0])
                pltpu.sync_copy(ei_hbm.at[1, pl.ds(ebase + r * _CH, _CH)],
                                dst_v.at[r, 0])

        @pl.when(last)
        def _():
            for r in range(6):
                pltpu.sync_copy(ei_hbm.at[0, pl.ds(ebase + r * _CH, _CH)],
                                src_v.at[r, 0])
                pltpu.sync_copy(ei_hbm.at[1, pl.ds(ebase + r * _CH, _CH)],
                                dst_v.at[r, 0])
            pltpu.sync_copy(ei_hbm.at[0, pl.ds(ebase + 6 * _CH, 512)],
                            src_v.at[6, 0, pl.ds(0, 512)])
            pltpu.sync_copy(ei_hbm.at[1, pl.ds(ebase + 6 * _CH, 512)],
                            dst_v.at[6, 0, pl.ds(0, 512)])

            def _pad(p, _):
                q = 6 * _CH + 512 + p * _L
                r = q // _CH
                c = q - r * _CH
                v = _N + lax.rem(lax.iota(jnp.int32, _L) + p * _L,
                                 _NPAD - _N)
                src_v[r, 0, pl.ds(c, _L)] = v
                dst_v[r, 0, pl.ds(c, _L)] = v
                return 0
            lax.fori_loop(0, (4 * _CH - 512) // _L, _pad, 0)

        pltpu.sync_copy(x_hbm, x_v)

        def _zero_sl(i, _):
            sl_v[pl.ds(i * _L, _L)] = jnp.zeros((_L,), jnp.float32)
            return 0
        lax.fori_loop(0, _NPT // _L, _zero_sl, 0)
        pltpu.sync_copy(sl_v, acc_sh.at[nsl])

        def _ones(j, _):
            r = j // (_CH // _L)
            c = j % (_CH // _L)
            w_v[r, 0, pl.ds(c * _L, _L)] = jnp.ones((_L,), jnp.float32)
            return 0
        lax.fori_loop(0, _RS * (_CH // _L), _ones, 0)

        plsc.subcore_barrier()

        for r in range(_RS):
            pltpu.sync_copy(w_v.at[r, 0], acc_sh.at[src_v.at[r, 0]], add=True)

        plsc.subcore_barrier()

        pltpu.sync_copy(acc_sh.at[nsl], sl_v)

        def _dinv(i, _):
            d = sl_v[pl.ds(i * _L, _L)]
            y = _rsqrt16(d)
            sl_v[pl.ds(i * _L, _L)] = jnp.where(d > 0.5, y, 0.0)
            return 0
        lax.fori_loop(0, _NPT // _L, _dinv, 0)
        pltpu.sync_copy(sl_v, aux_sh.at[nsl])
        lax.fori_loop(0, _NPT // _L, _zero_sl, 0)
        pltpu.sync_copy(sl_v, acc_sh.at[nsl])

        plsc.subcore_barrier()

        pltpu.sync_copy(aux_sh, dinv_v)

        def _wval(j, _):
            r = j // (_CH // _L)
            c = j % (_CH // _L)
            cs = pl.ds(c * _L, _L)
            s16 = src_v[r, 0, cs]
            d16 = dst_v[r, 0, cs]
            dvs = plsc.load_gather(dinv_v, [s16])
            dvd = plsc.load_gather(dinv_v, [d16])
            x16 = plsc.load_gather(x_v, [s16])
            w16 = -(dvs * dvd)
            w_v[r, 0, cs] = w16
            val_v[r, 0, cs] = w16 * x16
            return 0
        lax.fori_loop(0, _RS * (_CH // _L), _wval, 0)

        for r in range(_RS):
            pltpu.sync_copy(val_v.at[r, 0], acc_sh.at[dst_v.at[r, 0]], add=True)

        plsc.subcore_barrier()

        pltpu.sync_copy(acc_sh, t1_v)
        pltpu.sync_copy(acc_sh.at[nsl], t1_hbm.at[nsl])
        lax.fori_loop(0, _NPT // _L, _zero_sl, 0)
        pltpu.sync_copy(sl_v, aux_sh.at[nsl])

        plsc.subcore_barrier()

        def _val2(j, _):
            r = j // (_CH // _L)
            c = j % (_CH // _L)
            cs = pl.ds(c * _L, _L)
            s16 = src_v[r, 0, cs]
            t16 = plsc.load_gather(t1_v, [s16])
            val_v[r, 0, cs] = 2.0 * w_v[r, 0, cs] * t16
            return 0
        lax.fori_loop(0, _RS * (_CH // _L), _val2, 0)
        for r in range(_RS):
            pltpu.sync_copy(val_v.at[r, 0], aux_sh.at[dst_v.at[r, 0]], add=True)

        plsc.subcore_barrier()

        pltpu.sync_copy(aux_sh.at[nsl], sl_v)

        def _t2(i, _):
            ds = pl.ds(tid * _NPT + i * _L, _L)
            sl_v[pl.ds(i * _L, _L)] = sl_v[pl.ds(i * _L, _L)] - x_v[ds]
            return 0
        lax.fori_loop(0, _NPT // _L, _t2, 0)
        pltpu.sync_copy(sl_v, t2_hbm.at[nsl])

    return sc_kernel(x_pad, ei)


_NB = 400
_NBLK = _N // _NB


def _tc_body(t_ref, wr_ref, c_ref, bcb_ref, bfc_ref, out_ref, acc_ref):
    j = pl.program_id(0)

    @pl.when(j == 0)
    def _():
        acc_ref[...] = jnp.zeros_like(acc_ref)

    tb = t_ref[...]
    h = jnp.dot(tb, c_ref[...], preferred_element_type=jnp.float32)
    h = jnp.maximum(h + bcb_ref[...], 0.0)
    w3 = wr_ref[...]
    for d in range(_DOUT):
        acc_ref[d:d + 1, :] += jnp.sum(w3[d] * h, axis=0, keepdims=True)

    @pl.when(j == _NBLK - 1)
    def _():
        s = jnp.sum(acc_ref[0:_DOUT, :], axis=1, keepdims=True)
        y = s + bfc_ref[...]
        m = jnp.max(y, axis=0, keepdims=True)
        z = y - m
        lse = jnp.log(jnp.sum(jnp.exp(z), axis=0, keepdims=True))
        out_ref[...] = z - lse


def kernel(x, edge_index, W_cheb, b_cheb, W_fc, b_fc):
    x1 = x[:, 0]
    x_pad = jnp.pad(x1, (0, _NPAD - _N))

    t1, t2 = _sc_compute(x_pad, edge_index)

    tmat = jnp.concatenate(
        [x, t1[:_N, None], t2[:_N, None],
         jnp.zeros((_N, 5), jnp.float32)], axis=1)
    wr = W_fc.reshape(_DOUT, _N, _G)
    cmat = jnp.zeros((8, _G), jnp.float32).at[:3].set(W_cheb.reshape(3, _G))
    bcb = b_cheb.reshape(1, _G)
    bfc = b_fc.reshape(_DOUT, 1)

    out = pl.pallas_call(
        _tc_body,
        grid=(_NBLK,),
        in_specs=[
            pl.BlockSpec((_NB, 8), lambda j: (j, 0)),
            pl.BlockSpec((_DOUT, _NB, _G), lambda j: (0, j, 0)),
            pl.BlockSpec((8, _G), lambda j: (0, 0)),
            pl.BlockSpec((1, _G), lambda j: (0, 0)),
            pl.BlockSpec((_DOUT, 1), lambda j: (0, 0)),
        ],
        out_specs=pl.BlockSpec((_DOUT, 1), lambda j: (0, 0)),
        out_shape=jax.ShapeDtypeStruct((_DOUT, 1), jnp.float32),
        scratch_shapes=[pltpu.VMEM((16, _G), jnp.float32)],
        compiler_params=pltpu.CompilerParams(
            dimension_semantics=("arbitrary",)),
    )(tmat, wr, cmat, bcb, bfc)
    return out[:, 0]


# speedup vs baseline: 2.4825x; 2.4648x over previous
import functools

import jax
import jax.numpy as jnp
from jax import lax
from jax.experimental import pallas as pl
from jax.experimental.pallas import tpu as pltpu
from jax.experimental.pallas import tpu_sc as plsc

_N = 10000
_E = 320000
_G = 128
_DOUT = 10

_NTILES = 16
_NPAD = 10240
_NPT = _NPAD // _NTILES
_ECR = _E // _NTILES
_ECP = 20480
_ELAST = 12800
_NCHK = 4
_CHK = _ECP // _NCHK
_L = 16


def _rsqrt16(x):
    i = plsc.bitcast(x, jnp.int32)
    i = 0x5F3759DF - lax.shift_right_arithmetic(i, 1)
    y = plsc.bitcast(i, jnp.float32)
    for _ in range(3):
        y = y * (1.5 - 0.5 * x * y * y)
    return y


def _sc_compute(x_pad, ei):
    mesh = plsc.VectorSubcoreMesh(
        core_axis_name="c", subcore_axis_name="s", num_cores=1,
        num_subcores=_NTILES)

    @functools.partial(
        pl.kernel,
        out_type=(
            jax.ShapeDtypeStruct((_NPAD,), jnp.float32),
            jax.ShapeDtypeStruct((_NPAD,), jnp.float32),
        ),
        mesh=mesh,
        scratch_types=[
            pltpu.VMEM((_ECP,), jnp.int32),
            pltpu.VMEM((_ECP,), jnp.int32),
            pltpu.VMEM((_ECP,), jnp.float32),
            pltpu.VMEM((_NPAD,), jnp.float32),
            pltpu.VMEM((_NPAD,), jnp.float32),
            pltpu.VMEM((_NPAD,), jnp.float32),
            pltpu.VMEM((_NPT,), jnp.float32),
            pltpu.VMEM((_NPT,), jnp.float32),
            pltpu.VMEM_SHARED((_NPAD,), jnp.float32),
            pltpu.VMEM_SHARED((_NPAD,), jnp.float32),
            pltpu.VMEM_SHARED((_NPAD,), jnp.float32),
            pltpu.SemaphoreType.DMA,
        ],
        compiler_params=pltpu.CompilerParams(needs_layout_passes=False),
    )
    def sc_kernel(x_hbm, ei_hbm, t1_hbm, t2_hbm,
                  src_v, dst_v, val_v, x_v, dinv_v, g_v, sl_v, sl2_v,
                  acc_sh, dinv_sh, g_sh, sem):
        tid = lax.axis_index("s")
        nsl = pl.ds(tid * _NPT, _NPT)
        ebase = tid * _ECP
        last = tid == _NTILES - 1

        @pl.when(jnp.logical_not(last))
        def _():
            pltpu.async_copy(ei_hbm.at[0, pl.ds(ebase, _ECP)], src_v, sem)
            pltpu.async_copy(ei_hbm.at[1, pl.ds(ebase, _ECP)], dst_v, sem)

        @pl.when(last)
        def _():
            pltpu.async_copy(ei_hbm.at[0, pl.ds(ebase, _ELAST)],
                             src_v.at[pl.ds(0, _ELAST)], sem)
            pltpu.async_copy(ei_hbm.at[1, pl.ds(ebase, _ELAST)],
                             dst_v.at[pl.ds(0, _ELAST)], sem)

        pltpu.async_copy(x_hbm, x_v, sem)

        @plsc.parallel_loop(0, _ECP, _L, unroll=8)
        def _ones(e):
            val_v[pl.ds(e, _L)] = jnp.ones((_L,), jnp.float32)

        @pl.when(jnp.logical_not(last))
        def _():
            pltpu.make_async_copy(ei_hbm.at[0, pl.ds(ebase, _ECP)], src_v,
                                  sem).wait()
            pltpu.make_async_copy(ei_hbm.at[1, pl.ds(ebase, _ECP)], dst_v,
                                  sem).wait()

        @pl.when(last)
        def _():
            pltpu.make_async_copy(ei_hbm.at[0, pl.ds(ebase, _ELAST)],
                                  src_v.at[pl.ds(0, _ELAST)], sem).wait()
            pltpu.make_async_copy(ei_hbm.at[1, pl.ds(ebase, _ELAST)],
                                  dst_v.at[pl.ds(0, _ELAST)], sem).wait()

            @plsc.parallel_loop(0, _ECP - _ELAST, _L, unroll=8)
            def _pad(p):
                v = _N + lax.rem(lax.iota(jnp.int32, _L) + p, _NPAD - _N)
                src_v[pl.ds(_ELAST + p, _L)] = v
                dst_v[pl.ds(_ELAST + p, _L)] = v

        pltpu.make_async_copy(x_hbm, x_v, sem).wait()

        def _zero_sl():
            @plsc.parallel_loop(0, _NPT, _L, unroll=8)
            def _z(i):
                sl_v[pl.ds(i, _L)] = jnp.zeros((_L,), jnp.float32)
        _zero_sl()
        pltpu.sync_copy(sl_v, acc_sh.at[nsl])

        plsc.subcore_barrier()

        pltpu.sync_copy(val_v, acc_sh.at[src_v], add=True)

        plsc.subcore_barrier()

        pltpu.sync_copy(acc_sh.at[nsl], sl_v)

        @plsc.parallel_loop(0, _NPT, _L, unroll=4)
        def _dinv(i):
            cs = pl.ds(i, _L)
            d = sl_v[cs]
            y = jnp.where(d > 0.5, _rsqrt16(d), 0.0)
            sl_v[cs] = y
            sl2_v[cs] = -y * x_v[pl.ds(tid * _NPT + i, _L)]
        pltpu.sync_copy(sl_v, dinv_sh.at[nsl])
        pltpu.sync_copy(sl2_v, g_sh.at[nsl])
        _zero_sl()
        pltpu.sync_copy(sl_v, acc_sh.at[nsl])

        plsc.subcore_barrier()

        pltpu.sync_copy(dinv_sh, dinv_v)
        pltpu.sync_copy(g_sh, g_v)

        for ch in range(_NCHK):
            c0 = ch * _CHK

            @plsc.parallel_loop(c0, c0 + _CHK, _L, unroll=8)
            def _val1(e):
                cs = pl.ds(e, _L)
                val_v[cs] = (plsc.load_gather(g_v, [src_v[cs]])
                             * plsc.load_gather(dinv_v, [dst_v[cs]]))
            pltpu.async_copy(val_v.at[pl.ds(c0, _CHK)],
                             acc_sh.at[dst_v.at[pl.ds(c0, _CHK)]],
                             sem, add=True)
        for ch in range(_NCHK):
            c0 = ch * _CHK
            pltpu.make_async_copy(val_v.at[pl.ds(c0, _CHK)],
                                  acc_sh.at[dst_v.at[pl.ds(c0, _CHK)]],
                                  sem).wait()

        plsc.subcore_barrier()

        pltpu.sync_copy(acc_sh.at[nsl], sl_v)
        pltpu.sync_copy(sl_v, t1_hbm.at[nsl])

        @plsc.parallel_loop(0, _NPT, _L, unroll=4)
        def _bfill(i):
            cs = pl.ds(i, _L)
            sl2_v[cs] = -2.0 * sl_v[cs] * dinv_v[pl.ds(tid * _NPT + i, _L)]
        pltpu.sync_copy(sl2_v, g_sh.at[nsl])
        _zero_sl()
        pltpu.sync_copy(sl_v, dinv_sh.at[nsl])

        plsc.subcore_barrier()

        pltpu.sync_copy(g_sh, g_v)

        for ch in range(_NCHK):
            c0 = ch * _CHK

            @plsc.parallel_loop(c0, c0 + _CHK, _L, unroll=8)
            def _val2(e):
                cs = pl.ds(e, _L)
                val_v[cs] = (plsc.load_gather(g_v, [src_v[cs]])
                             * plsc.load_gather(dinv_v, [dst_v[cs]]))
            pltpu.async_copy(val_v.at[pl.ds(c0, _CHK)],
                             dinv_sh.at[dst_v.at[pl.ds(c0, _CHK)]],
                             sem, add=True)
        for ch in range(_NCHK):
            c0 = ch * _CHK
            pltpu.make_async_copy(val_v.at[pl.ds(c0, _CHK)],
                                  dinv_sh.at[dst_v.at[pl.ds(c0, _CHK)]],
                                  sem).wait()

        plsc.subcore_barrier()

        pltpu.sync_copy(dinv_sh.at[nsl], sl_v)

        @plsc.parallel_loop(0, _NPT, _L, unroll=4)
        def _t2(i):
            cs = pl.ds(i, _L)
            sl_v[cs] = sl_v[cs] - x_v[pl.ds(tid * _NPT + i, _L)]
        pltpu.sync_copy(sl_v, t2_hbm.at[nsl])

    return sc_kernel(x_pad, ei)


_NB = 1000
_NBLK = _N // _NB


def _tc_body(t_ref, wa_ref, wb_ref, c_ref, bcb_ref, bfc_ref, out_ref,
             acc_ref):
    j = pl.program_id(0)

    @pl.when(j == 0)
    def _():
        acc_ref[...] = jnp.zeros_like(acc_ref)

    tb = t_ref[...]
    h = jnp.dot(tb, c_ref[...], preferred_element_type=jnp.float32)
    h = jnp.maximum(h + bcb_ref[...], 0.0)
    hf = h.reshape(1, _NB * _G)
    pa = lax.dot_general(wa_ref[...], hf, (((1,), (1,)), ((), ())),
                         preferred_element_type=jnp.float32)
    pb = lax.dot_general(wb_ref[...], hf, (((1,), (1,)), ((), ())),
                         preferred_element_type=jnp.float32)
    acc_ref[0:8, 0:1] += pa
    acc_ref[8:_DOUT, 0:1] += pb

    @pl.when(j == _NBLK - 1)
    def _():
        y = acc_ref[0:_DOUT, 0:1] + bfc_ref[...]
        m = jnp.max(y, axis=0, keepdims=True)
        z = y - m
        lse = jnp.log(jnp.sum(jnp.exp(z), axis=0, keepdims=True))
        out_ref[...] = z - lse


def kernel(x, edge_index, W_cheb, b_cheb, W_fc, b_fc):
    x1 = x[:, 0]
    x_pad = jnp.pad(x1, (0, _NPAD - _N))

    t1, t2 = _sc_compute(x_pad, edge_index)

    tmat = jnp.concatenate(
        [x, t1[:_N, None], t2[:_N, None],
         jnp.zeros((_N, 5), jnp.float32)], axis=1)
    W_tail = W_fc[8:_DOUT]
    cmat = jnp.zeros((8, _G), jnp.float32).at[:3].set(W_cheb.reshape(3, _G))
    bcb = b_cheb.reshape(1, _G)
    bfc = b_fc.reshape(_DOUT, 1)

    out = pl.pallas_call(
        _tc_body,
        grid=(_NBLK,),
        in_specs=[
            pl.BlockSpec((_NB, 8), lambda j: (j, 0)),
            pl.BlockSpec((8, _NB * _G), lambda j: (0, j)),
            pl.BlockSpec((2, _NB * _G), lambda j: (0, j)),
            pl.BlockSpec((8, _G), lambda j: (0, 0)),
            pl.BlockSpec((1, _G), lambda j: (0, 0)),
            pl.BlockSpec((_DOUT, 1), lambda j: (0, 0)),
        ],
        out_specs=pl.BlockSpec((_DOUT, 1), lambda j: (0, 0)),
        out_shape=jax.ShapeDtypeStruct((_DOUT, 1), jnp.float32),
        scratch_shapes=[pltpu.VMEM((16, _G), jnp.float32)],
        compiler_params=pltpu.CompilerParams(
            dimension_semantics=("arbitrary",),
            vmem_limit_bytes=100 * 1024 * 1024),
    )(tmat, W_fc, W_tail, cmat, bcb, bfc)
    return out[:, 0]
